# Initial kernel scaffold; baseline (speedup 1.0000x reference)
import jax
import jax.numpy as jnp
from jax.experimental import pallas as pl


def kernel(x, edge_index, edge_weight, W1, b1, W2, b2, Wc1, bc1, Wc2, bc2):
    raise NotImplementedError("write your pallas kernel here")


# speedup vs baseline: 3.8781x; 3.8781x over previous
import functools

import jax
import jax.numpy as jnp
from jax import lax
from jax.experimental import pallas as pl
from jax.experimental.pallas import tpu as pltpu
from jax.experimental.pallas import tpu_sc as plsc

N = 10000
E = 320000
SLOPE = 0.2

NC = 2
NS = 16
NW = NC * NS
CHUNK = 80
EDGES_PER_W = E // NW
CHUNKS_PER_W = EDGES_PER_W // CHUNK
ROWS_PER_TILE = 632
NPAD = NS * ROWS_PER_TILE


def _make_spmm(H):
    mesh = plsc.VectorSubcoreMesh(core_axis_name="c", subcore_axis_name="s")

    @functools.partial(
        pl.kernel,
        mesh=mesh,
        compiler_params=pltpu.CompilerParams(use_tc_tiling_on_sc=False),
        out_type=jax.ShapeDtypeStruct((2 * NPAD, H), jnp.float32),
        scratch_types=[
            pltpu.VMEM((CHUNK,), jnp.int32),
            pltpu.VMEM((CHUNK,), jnp.int32),
            pltpu.VMEM((CHUNK + 16,), jnp.float32),
            pltpu.VMEM((CHUNK, H), jnp.float32),
            pltpu.VMEM((ROWS_PER_TILE, H), jnp.float32),
            pltpu.VMEM_SHARED((NPAD, H), jnp.float32),
            pltpu.SemaphoreType.DMA,
        ],
    )
    def spmm(sup_hbm, col_hbm, row_hbm, w_hbm, out_hbm,
             col_v, row_v, w_v, rows_v, stage_v, acc_sh, sem):
        cid = lax.axis_index("c")
        sid = lax.axis_index("s")

        def zrow(r, carry):
            for hb in range(H // 16):
                stage_v[r, pl.ds(hb * 16, 16)] = jnp.zeros((16,), jnp.float32)
            return carry
        lax.fori_loop(0, ROWS_PER_TILE, zrow, 0)
        pltpu.sync_copy(stage_v,
                        acc_sh.at[pl.ds(sid * ROWS_PER_TILE, ROWS_PER_TILE)])
        plsc.subcore_barrier()

        wid = sid * NC + cid
        base_edge = wid * EDGES_PER_W

        def chunk_body(j, carry):
            eb = base_edge + j * CHUNK
            pltpu.sync_copy(col_hbm.at[pl.ds(eb, CHUNK)], col_v)
            pltpu.sync_copy(row_hbm.at[pl.ds(eb, CHUNK)], row_v)
            pltpu.sync_copy(w_hbm.at[pl.ds(eb, CHUNK)], w_v.at[pl.ds(0, CHUNK)])
            pltpu.async_copy(sup_hbm.at[col_v], rows_v, sem).wait()

            def edge_body(e, c2):
                wb = w_v[pl.ds(e, 16)][0]
                for hb in range(H // 16):
                    sl = pl.ds(hb * 16, 16)
                    rows_v[e, sl] = rows_v[e, sl] * wb
                return c2
            lax.fori_loop(0, CHUNK, edge_body, 0)

            pltpu.sync_copy(rows_v, acc_sh.at[row_v], add=True)
            return carry
        lax.fori_loop(0, CHUNKS_PER_W, chunk_body, 0)

        plsc.subcore_barrier()
        pltpu.sync_copy(acc_sh.at[pl.ds(sid * ROWS_PER_TILE, ROWS_PER_TILE)],
                        stage_v)
        pltpu.sync_copy(stage_v,
                        out_hbm.at[pl.ds(cid * NPAD + sid * ROWS_PER_TILE,
                                         ROWS_PER_TILE)])

    return spmm


_spmm64 = _make_spmm(64)
_spmm32 = _make_spmm(32)

_BLK = 1000


def _mm_body(x_ref, w_ref, o_ref):
    o_ref[...] = jnp.dot(x_ref[...], w_ref[...],
                         preferred_element_type=jnp.float32)


def _tc_mm(x, w):
    m, k = x.shape
    n = w.shape[1]
    return pl.pallas_call(
        _mm_body,
        grid=(m // _BLK,),
        in_specs=[pl.BlockSpec((_BLK, k), lambda i: (i, 0)),
                  pl.BlockSpec((k, n), lambda i: (0, 0))],
        out_specs=pl.BlockSpec((_BLK, n), lambda i: (i, 0)),
        out_shape=jax.ShapeDtypeStruct((m, n), jnp.float32),
    )(x, w)


def _leaky(h):
    return jnp.where(h > 0, h, SLOPE * h)


def _combine_mm_body(p_ref, b_ref, w_ref, o_ref):
    h = p_ref[0] + p_ref[1] + b_ref[...]
    h = _leaky(h)
    o_ref[...] = jnp.dot(h, w_ref[...], preferred_element_type=jnp.float32)


def _tc_combine_mm(parts, b, w):
    _, m, k = parts.shape
    n = w.shape[1]
    return pl.pallas_call(
        _combine_mm_body,
        grid=(m // _BLK,),
        in_specs=[pl.BlockSpec((2, _BLK, k), lambda i: (0, i, 0)),
                  pl.BlockSpec((1, k), lambda i: (0, 0)),
                  pl.BlockSpec((k, n), lambda i: (0, 0))],
        out_specs=pl.BlockSpec((_BLK, n), lambda i: (i, 0)),
        out_shape=jax.ShapeDtypeStruct((m, n), jnp.float32),
    )(parts, b, w)


def _head_body(p_ref, b2_ref, wc1_ref, bc1_ref, wc2_ref, bc2_ref, o_ref):
    h = p_ref[0] + p_ref[1] + b2_ref[...]
    h = _leaky(h)
    h = _leaky(jnp.dot(h, wc1_ref[...], preferred_element_type=jnp.float32)
               + bc1_ref[...])
    o_ref[...] = jnp.dot(h, wc2_ref[...],
                         preferred_element_type=jnp.float32) + bc2_ref[...]


def _tc_head(parts, b2, wc1t, bc1, wc2t, bc2):
    _, m, k = parts.shape
    h3 = wc1t.shape[1]
    out = wc2t.shape[1]
    return pl.pallas_call(
        _head_body,
        grid=(m // _BLK,),
        in_specs=[pl.BlockSpec((2, _BLK, k), lambda i: (0, i, 0)),
                  pl.BlockSpec((1, k), lambda i: (0, 0)),
                  pl.BlockSpec((k, h3), lambda i: (0, 0)),
                  pl.BlockSpec((1, h3), lambda i: (0, 0)),
                  pl.BlockSpec((h3, out), lambda i: (0, 0)),
                  pl.BlockSpec((1, out), lambda i: (0, 0))],
        out_specs=pl.BlockSpec((_BLK, out), lambda i: (i, 0)),
        out_shape=jax.ShapeDtypeStruct((m, out), jnp.float32),
    )(parts, b2, wc1t, bc1, wc2t, bc2)


def kernel(x, edge_index, edge_weight, W1, b1, W2, b2, Wc1, bc1, Wc2, bc2):
    row = edge_index[0]
    col = edge_index[1]

    support1 = _tc_mm(x, W1)
    part1 = _spmm64(support1, col, row, edge_weight)
    part1 = part1.reshape(2, NPAD, 64)[:, :N]
    support2 = _tc_combine_mm(part1, b1.reshape(1, -1), W2)
    part2 = _spmm32(support2, col, row, edge_weight)
    part2 = part2.reshape(2, NPAD, 32)[:, :N]
    out = _tc_head(part2, b2.reshape(1, -1), Wc1.T, bc1.reshape(1, -1),
                   Wc2.T, bc2.reshape(1, -1))
    return out


# speedup vs baseline: 12.6387x; 3.2590x over previous
import functools

import jax
import jax.numpy as jnp
from jax import lax
from jax.experimental import pallas as pl
from jax.experimental.pallas import tpu as pltpu
from jax.experimental.pallas import tpu_sc as plsc

N = 10000
E = 320000
SLOPE = 0.2

NC = 2
NS = 16
NW = NC * NS
CHUNK = 80
EDGES_PER_W = E // NW
CHUNKS_PER_W = EDGES_PER_W // CHUNK
ROWS_PER_TILE = 632
NPAD = NS * ROWS_PER_TILE


_WCHUNK = 158


def _make_spmm(H):
    mesh = plsc.VectorSubcoreMesh(core_axis_name="c", subcore_axis_name="s")
    HB = H // 16
    NCH = CHUNKS_PER_W

    @functools.partial(
        pl.kernel,
        mesh=mesh,
        compiler_params=pltpu.CompilerParams(use_tc_tiling_on_sc=False),
        out_type=jax.ShapeDtypeStruct((2 * NPAD, H), jnp.float32),
        scratch_types=[
            pltpu.VMEM((NCH, CHUNK), jnp.int32),
            pltpu.VMEM((NCH, CHUNK), jnp.int32),
            pltpu.VMEM((NCH, CHUNK), jnp.float32),
            pltpu.VMEM((CHUNK, H), jnp.float32),
            pltpu.VMEM((CHUNK, H), jnp.float32),
            pltpu.VMEM((CHUNK, H), jnp.float32),
            pltpu.VMEM((CHUNK, H), jnp.float32),
            pltpu.VMEM((_WCHUNK, H), jnp.float32),
            pltpu.VMEM_SHARED((NPAD, H), jnp.float32),
            pltpu.SemaphoreType.DMA,
            pltpu.SemaphoreType.DMA,
            pltpu.SemaphoreType.DMA,
            pltpu.SemaphoreType.DMA,
        ],
    )
    def spmm(sup_hbm, col_hbm, row_hbm, w_hbm, out_hbm,
             col_l, row_l, w_l, gb0, gb1, sb0, sb1, bounce, acc_sh,
             gs0, gs1, ss0, ss1):
        cid = lax.axis_index("c")
        sid = lax.axis_index("s")
        wid = sid * NC + cid
        gbufs, sbufs = (gb0, gb1), (sb0, sb1)
        gsems, ssems = (gs0, gs1), (ss0, ss1)

        wbase = wid * NCH
        pltpu.sync_copy(col_hbm.at[pl.ds(wbase, NCH)], col_l)
        pltpu.sync_copy(row_hbm.at[pl.ds(wbase, NCH)], row_l)
        pltpu.sync_copy(w_hbm.at[pl.ds(wbase, NCH)], w_l)

        def zrow(r, carry):
            for hb in range(HB):
                bounce[r, pl.ds(hb * 16, 16)] = jnp.zeros((16,), jnp.float32)
            return carry
        lax.fori_loop(0, _WCHUNK, zrow, 0)
        tile_base = sid * ROWS_PER_TILE
        for q in range(ROWS_PER_TILE // _WCHUNK):
            pltpu.sync_copy(bounce,
                            acc_sh.at[pl.ds(tile_base + q * _WCHUNK, _WCHUNK)])
        plsc.subcore_barrier()

        def gather_start(b, j):
            pltpu.async_copy(sup_hbm.at[col_l.at[j]], gbufs[b], gsems[b])

        def gather_wait(b):
            pltpu.make_async_copy(sup_hbm.at[col_l.at[0]], gbufs[b],
                                  gsems[b]).wait()

        def scatter_start(b, j):
            pltpu.async_copy(sbufs[b], acc_sh.at[row_l.at[j]], ssems[b],
                             add=True)

        def scatter_wait(b):
            pltpu.make_async_copy(sbufs[b], acc_sh.at[row_l.at[0]],
                                  ssems[b]).wait()

        def scale(b, j):
            gb, sb = gbufs[b], sbufs[b]
            for g in range(CHUNK // 16):
                w16 = w_l[j, pl.ds(g * 16, 16)]
                for l in range(16):
                    e = g * 16 + l
                    wb = w16[l]
                    for hb in range(HB):
                        sl = pl.ds(hb * 16, 16)
                        sb[e, sl] = gb[e, sl] * wb

        gather_start(0, 0)
        gather_start(1, 1)

        def pair_body(i, carry):
            for b in range(2):
                j = 2 * i + b
                gather_wait(b)

                @pl.when(i > 0)
                def _():
                    scatter_wait(b)
                scale(b, j)
                nxt = j + 2

                @pl.when(nxt < NCH)
                def _():
                    gather_start(b, nxt)
                scatter_start(b, j)
            return carry
        lax.fori_loop(0, NCH // 2, pair_body, 0)

        jt = NCH - 1
        gather_wait(0)
        scatter_wait(0)
        scale(0, jt)
        scatter_start(0, jt)
        scatter_wait(0)
        scatter_wait(1)

        plsc.subcore_barrier()
        out_base = cid * NPAD + tile_base
        for q in range(ROWS_PER_TILE // _WCHUNK):
            pltpu.sync_copy(acc_sh.at[pl.ds(tile_base + q * _WCHUNK, _WCHUNK)],
                            bounce)
            pltpu.sync_copy(bounce,
                            out_hbm.at[pl.ds(out_base + q * _WCHUNK, _WCHUNK)])

    return spmm


_spmm64 = _make_spmm(64)
_spmm32 = _make_spmm(32)

_BLK = 1000


def _mm_body(x_ref, w_ref, o_ref):
    o_ref[...] = jnp.dot(x_ref[...], w_ref[...],
                         preferred_element_type=jnp.float32)


def _tc_mm(x, w):
    m, k = x.shape
    n = w.shape[1]
    return pl.pallas_call(
        _mm_body,
        grid=(m // _BLK,),
        in_specs=[pl.BlockSpec((_BLK, k), lambda i: (i, 0)),
                  pl.BlockSpec((k, n), lambda i: (0, 0))],
        out_specs=pl.BlockSpec((_BLK, n), lambda i: (i, 0)),
        out_shape=jax.ShapeDtypeStruct((m, n), jnp.float32),
    )(x, w)


def _leaky(h):
    return jnp.where(h > 0, h, SLOPE * h)


def _combine_mm_body(p_ref, b_ref, w_ref, o_ref):
    h = p_ref[0] + p_ref[1] + b_ref[...]
    h = _leaky(h)
    o_ref[...] = jnp.dot(h, w_ref[...], preferred_element_type=jnp.float32)


def _tc_combine_mm(parts, b, w):
    _, m, k = parts.shape
    n = w.shape[1]
    return pl.pallas_call(
        _combine_mm_body,
        grid=(m // _BLK,),
        in_specs=[pl.BlockSpec((2, _BLK, k), lambda i: (0, i, 0)),
                  pl.BlockSpec((1, k), lambda i: (0, 0)),
                  pl.BlockSpec((k, n), lambda i: (0, 0))],
        out_specs=pl.BlockSpec((_BLK, n), lambda i: (i, 0)),
        out_shape=jax.ShapeDtypeStruct((m, n), jnp.float32),
    )(parts, b, w)


def _head_body(p_ref, b2_ref, wc1_ref, bc1_ref, wc2_ref, bc2_ref, o_ref):
    h = p_ref[0] + p_ref[1] + b2_ref[...]
    h = _leaky(h)
    h = _leaky(jnp.dot(h, wc1_ref[...], preferred_element_type=jnp.float32)
               + bc1_ref[...])
    o_ref[...] = jnp.dot(h, wc2_ref[...],
                         preferred_element_type=jnp.float32) + bc2_ref[...]


def _tc_head(parts, b2, wc1t, bc1, wc2t, bc2):
    _, m, k = parts.shape
    h3 = wc1t.shape[1]
    out = wc2t.shape[1]
    return pl.pallas_call(
        _head_body,
        grid=(m // _BLK,),
        in_specs=[pl.BlockSpec((2, _BLK, k), lambda i: (0, i, 0)),
                  pl.BlockSpec((1, k), lambda i: (0, 0)),
                  pl.BlockSpec((k, h3), lambda i: (0, 0)),
                  pl.BlockSpec((1, h3), lambda i: (0, 0)),
                  pl.BlockSpec((h3, out), lambda i: (0, 0)),
                  pl.BlockSpec((1, out), lambda i: (0, 0))],
        out_specs=pl.BlockSpec((_BLK, out), lambda i: (i, 0)),
        out_shape=jax.ShapeDtypeStruct((m, out), jnp.float32),
    )(parts, b2, wc1t, bc1, wc2t, bc2)


def kernel(x, edge_index, edge_weight, W1, b1, W2, b2, Wc1, bc1, Wc2, bc2):
    row = edge_index[0].reshape(NW * CHUNKS_PER_W, CHUNK)
    col = edge_index[1].reshape(NW * CHUNKS_PER_W, CHUNK)
    edge_weight = edge_weight.reshape(NW * CHUNKS_PER_W, CHUNK)

    support1 = _tc_mm(x, W1)
    part1 = _spmm64(support1, col, row, edge_weight)
    part1 = part1.reshape(2, NPAD, 64)[:, :N]
    support2 = _tc_combine_mm(part1, b1.reshape(1, -1), W2)
    part2 = _spmm32(support2, col, row, edge_weight)
    part2 = part2.reshape(2, NPAD, 32)[:, :N]
    out = _tc_head(part2, b2.reshape(1, -1), Wc1.T, bc1.reshape(1, -1),
                   Wc2.T, bc2.reshape(1, -1))
    return out


# speedup vs baseline: 12.9715x; 1.0263x over previous
import functools

import jax
import jax.numpy as jnp
from jax import lax
from jax.experimental import pallas as pl
from jax.experimental.pallas import tpu as pltpu
from jax.experimental.pallas import tpu_sc as plsc

N = 10000
E = 320000
SLOPE = 0.2

NC = 2
NS = 16
NW = NC * NS
CHUNK = 80
EDGES_PER_W = E // NW
CHUNKS_PER_W = EDGES_PER_W // CHUNK
ROWS_PER_TILE = 632
NPAD = NS * ROWS_PER_TILE


_WCHUNK = 158


def _make_spmm(H):
    mesh = plsc.VectorSubcoreMesh(core_axis_name="c", subcore_axis_name="s")
    HB = H // 16
    NCH = CHUNKS_PER_W

    @functools.partial(
        pl.kernel,
        mesh=mesh,
        compiler_params=pltpu.CompilerParams(use_tc_tiling_on_sc=False),
        out_type=jax.ShapeDtypeStruct((2 * NPAD, H), jnp.float32),
        scratch_types=[
            pltpu.VMEM((NCH, CHUNK), jnp.int32),
            pltpu.VMEM((NCH, CHUNK), jnp.int32),
            pltpu.VMEM((NCH, CHUNK), jnp.float32),
            pltpu.VMEM((CHUNK, H), jnp.float32),
            pltpu.VMEM((CHUNK, H), jnp.float32),
            pltpu.VMEM((CHUNK, H), jnp.float32),
            pltpu.VMEM((CHUNK, H), jnp.float32),
            pltpu.VMEM((_WCHUNK, H), jnp.float32),
            pltpu.VMEM_SHARED((NPAD, H), jnp.float32),
            pltpu.SemaphoreType.DMA,
            pltpu.SemaphoreType.DMA,
            pltpu.SemaphoreType.DMA,
            pltpu.SemaphoreType.DMA,
        ],
    )
    def spmm(sup_hbm, col_hbm, row_hbm, w_hbm, out_hbm,
             col_l, row_l, w_l, gb0, gb1, sb0, sb1, bounce, acc_sh,
             gs0, gs1, ss0, ss1):
        cid = lax.axis_index("c")
        sid = lax.axis_index("s")
        wid = sid * NC + cid
        gbufs, sbufs = (gb0, gb1), (sb0, sb1)
        gsems, ssems = (gs0, gs1), (ss0, ss1)

        wbase = wid * NCH
        pltpu.sync_copy(col_hbm.at[pl.ds(wbase, NCH)], col_l)
        pltpu.sync_copy(row_hbm.at[pl.ds(wbase, NCH)], row_l)
        pltpu.sync_copy(w_hbm.at[pl.ds(wbase, NCH)], w_l)

        def zrow(r, carry):
            for hb in range(HB):
                bounce[r, pl.ds(hb * 16, 16)] = jnp.zeros((16,), jnp.float32)
            return carry
        lax.fori_loop(0, _WCHUNK, zrow, 0)
        tile_base = sid * ROWS_PER_TILE
        for q in range(ROWS_PER_TILE // _WCHUNK):
            pltpu.sync_copy(bounce,
                            acc_sh.at[pl.ds(tile_base + q * _WCHUNK, _WCHUNK)])
        plsc.subcore_barrier()

        def gather_start(b, j):
            pltpu.async_copy(sup_hbm.at[col_l.at[j]], gbufs[b], gsems[b])

        def gather_wait(b):
            pltpu.make_async_copy(sup_hbm.at[col_l.at[0]], gbufs[b],
                                  gsems[b]).wait()

        def scatter_start(b, j):
            pltpu.async_copy(sbufs[b], acc_sh.at[row_l.at[j]], ssems[b],
                             add=True)

        def scatter_wait(b):
            pltpu.make_async_copy(sbufs[b], acc_sh.at[row_l.at[0]],
                                  ssems[b]).wait()

        def scale(b, j):
            gb, sb = gbufs[b], sbufs[b]
            for g in range(CHUNK // 16):
                w16 = w_l[j, pl.ds(g * 16, 16)]
                for l in range(16):
                    e = g * 16 + l
                    wb = w16[l]
                    for hb in range(HB):
                        sl = pl.ds(hb * 16, 16)
                        sb[e, sl] = gb[e, sl] * wb

        gather_start(0, 0)
        gather_start(1, 1)

        def pair_body(i, carry):
            for b in range(2):
                j = 2 * i + b
                gather_wait(b)

                @pl.when(i > 0)
                def _():
                    scatter_wait(b)
                scale(b, j)
                nxt = j + 2

                @pl.when(nxt < NCH)
                def _():
                    gather_start(b, nxt)
                scatter_start(b, j)
            return carry
        lax.fori_loop(0, NCH // 2, pair_body, 0)

        jt = NCH - 1
        gather_wait(0)
        scatter_wait(0)
        scale(0, jt)
        scatter_start(0, jt)
        scatter_wait(0)
        scatter_wait(1)

        plsc.subcore_barrier()
        out_base = cid * NPAD + tile_base
        for q in range(ROWS_PER_TILE // _WCHUNK):
            pltpu.sync_copy(acc_sh.at[pl.ds(tile_base + q * _WCHUNK, _WCHUNK)],
                            bounce)
            pltpu.sync_copy(bounce,
                            out_hbm.at[pl.ds(out_base + q * _WCHUNK, _WCHUNK)])

    return spmm


_spmm64 = _make_spmm(64)
_spmm32 = _make_spmm(32)

_BLK = 1000
_BLKP = 632


def _mm_body(x_ref, w_ref, o_ref):
    o_ref[...] = jnp.dot(x_ref[...], w_ref[...],
                         preferred_element_type=jnp.float32)


def _tc_mm(x, w):
    m, k = x.shape
    n = w.shape[1]
    return pl.pallas_call(
        _mm_body,
        grid=(m // _BLK,),
        in_specs=[pl.BlockSpec((_BLK, k), lambda i: (i, 0)),
                  pl.BlockSpec((k, n), lambda i: (0, 0))],
        out_specs=pl.BlockSpec((_BLK, n), lambda i: (i, 0)),
        out_shape=jax.ShapeDtypeStruct((m, n), jnp.float32),
    )(x, w)


def _leaky(h):
    return jnp.where(h > 0, h, SLOPE * h)


def _combine_mm_body(p_ref, b_ref, w_ref, o_ref):
    h = p_ref[0] + p_ref[1] + b_ref[...]
    h = _leaky(h)
    o_ref[...] = jnp.dot(h, w_ref[...], preferred_element_type=jnp.float32)


def _tc_combine_mm(parts, b, w):
    _, m, k = parts.shape
    n = w.shape[1]
    return pl.pallas_call(
        _combine_mm_body,
        grid=(m // _BLKP,),
        in_specs=[pl.BlockSpec((2, _BLKP, k), lambda i: (0, i, 0)),
                  pl.BlockSpec((1, k), lambda i: (0, 0)),
                  pl.BlockSpec((k, n), lambda i: (0, 0))],
        out_specs=pl.BlockSpec((_BLKP, n), lambda i: (i, 0)),
        out_shape=jax.ShapeDtypeStruct((m, n), jnp.float32),
    )(parts, b, w)


def _head_body(p_ref, b2_ref, wc1_ref, bc1_ref, wc2_ref, bc2_ref, o_ref):
    h = p_ref[0] + p_ref[1] + b2_ref[...]
    h = _leaky(h)
    h = _leaky(jnp.dot(h, wc1_ref[...], preferred_element_type=jnp.float32)
               + bc1_ref[...])
    o_ref[...] = jnp.dot(h, wc2_ref[...],
                         preferred_element_type=jnp.float32) + bc2_ref[...]


def _tc_head(parts, b2, wc1t, bc1, wc2t, bc2):
    _, m, k = parts.shape
    h3 = wc1t.shape[1]
    out = wc2t.shape[1]
    return pl.pallas_call(
        _head_body,
        grid=(m // _BLKP,),
        in_specs=[pl.BlockSpec((2, _BLKP, k), lambda i: (0, i, 0)),
                  pl.BlockSpec((1, k), lambda i: (0, 0)),
                  pl.BlockSpec((k, h3), lambda i: (0, 0)),
                  pl.BlockSpec((1, h3), lambda i: (0, 0)),
                  pl.BlockSpec((h3, out), lambda i: (0, 0)),
                  pl.BlockSpec((1, out), lambda i: (0, 0))],
        out_specs=pl.BlockSpec((_BLKP, out), lambda i: (i, 0)),
        out_shape=jax.ShapeDtypeStruct((m, out), jnp.float32),
    )(parts, b2, wc1t, bc1, wc2t, bc2)


def kernel(x, edge_index, edge_weight, W1, b1, W2, b2, Wc1, bc1, Wc2, bc2):
    row = edge_index[0].reshape(NW * CHUNKS_PER_W, CHUNK)
    col = edge_index[1].reshape(NW * CHUNKS_PER_W, CHUNK)
    edge_weight = edge_weight.reshape(NW * CHUNKS_PER_W, CHUNK)

    support1 = _tc_mm(x, W1)
    part1 = _spmm64(support1, col, row, edge_weight).reshape(2, NPAD, 64)
    support2 = _tc_combine_mm(part1, b1.reshape(1, -1), W2)
    part2 = _spmm32(support2, col, row, edge_weight).reshape(2, NPAD, 32)
    out = _tc_head(part2, b2.reshape(1, -1), Wc1.T, bc1.reshape(1, -1),
                   Wc2.T, bc2.reshape(1, -1))
    return out[:N]


# speedup vs baseline: 14.8209x; 1.1426x over previous
import functools

import jax
import jax.numpy as jnp
from jax import lax
from jax.experimental import pallas as pl
from jax.experimental.pallas import tpu as pltpu
from jax.experimental.pallas import tpu_sc as plsc

N = 10000
E = 320000
SLOPE = 0.2

NC = 2
NS = 16
NW = NC * NS
CHUNK = 80
EDGES_PER_W = E // NW
CHUNKS_PER_W = EDGES_PER_W // CHUNK
RPT = N // NS
WCH = RPT // 5
NBUF = 4
assert CHUNKS_PER_W % NBUF == 1


def _make_spmm(H):
    mesh = plsc.VectorSubcoreMesh(core_axis_name="c", subcore_axis_name="s")
    HB = H // 16
    NCH = CHUNKS_PER_W

    @functools.partial(
        pl.kernel,
        mesh=mesh,
        compiler_params=pltpu.CompilerParams(use_tc_tiling_on_sc=False),
        out_type=jax.ShapeDtypeStruct((2 * N, H), jnp.float32),
        scratch_types=(
            [pltpu.VMEM((NCH, CHUNK), jnp.int32),
             pltpu.VMEM((NCH, CHUNK), jnp.int32),
             pltpu.VMEM((NCH, CHUNK), jnp.float32)]
            + [pltpu.VMEM((CHUNK, H), jnp.float32)] * NBUF
            + [pltpu.VMEM((CHUNK, H), jnp.float32)] * NBUF
            + [pltpu.VMEM((WCH, H), jnp.float32),
               pltpu.VMEM_SHARED((N, H), jnp.float32)]
            + [pltpu.SemaphoreType.DMA] * (2 * NBUF)
        ),
    )
    def spmm(sup_hbm, cr_hbm, w_hbm, out_hbm, col_l, row_l, w_l, *rest):
        gbufs = rest[:NBUF]
        sbufs = rest[NBUF:2 * NBUF]
        bounce = rest[2 * NBUF]
        acc_sh = rest[2 * NBUF + 1]
        gsems = rest[2 * NBUF + 2:3 * NBUF + 2]
        ssems = rest[3 * NBUF + 2:4 * NBUF + 2]
        cid = lax.axis_index("c")
        sid = lax.axis_index("s")
        wid = sid * NC + cid
        s0 = sid * RPT

        wbase = wid * NCH
        pltpu.sync_copy(cr_hbm.at[pl.ds(wbase, NCH)], col_l)
        pltpu.sync_copy(w_hbm.at[pl.ds(wbase, NCH)], w_l)

        def unpack_row(j, carry):
            for g in range(CHUNK // 16):
                sl = pl.ds(g * 16, 16)
                pk = col_l[j, sl]
                row_l[j, sl] = lax.shift_right_logical(pk, 16)
                col_l[j, sl] = lax.bitwise_and(pk, 0xFFFF)
            return carry
        lax.fori_loop(0, NCH, unpack_row, 0)

        def zrow(r, carry):
            for hb in range(HB):
                bounce[r, pl.ds(hb * 16, 16)] = jnp.zeros((16,), jnp.float32)
            return carry
        lax.fori_loop(0, WCH, zrow, 0)
        for q in range(RPT // WCH):
            pltpu.sync_copy(bounce, acc_sh.at[pl.ds(s0 + q * WCH, WCH)])
        plsc.subcore_barrier()

        def gather_start(b, j):
            pltpu.async_copy(sup_hbm.at[col_l.at[j]], gbufs[b], gsems[b])

        def gather_wait(b):
            pltpu.make_async_copy(sup_hbm.at[col_l.at[0]], gbufs[b],
                                  gsems[b]).wait()

        def scatter_start(b, j):
            pltpu.async_copy(sbufs[b], acc_sh.at[row_l.at[j]], ssems[b],
                             add=True)

        def scatter_wait(b):
            pltpu.make_async_copy(sbufs[b], acc_sh.at[row_l.at[0]],
                                  ssems[b]).wait()

        def scale(b, j):
            gb, sb = gbufs[b], sbufs[b]
            for g in range(CHUNK // 16):
                w16 = w_l[j, pl.ds(g * 16, 16)]
                for l in range(16):
                    e = g * 16 + l
                    wb = w16[l]
                    for hb in range(HB):
                        sl = pl.ds(hb * 16, 16)
                        sb[e, sl] = gb[e, sl] * wb

        for b in range(NBUF):
            gather_start(b, b)

        def round_body(i, carry):
            for b in range(NBUF):
                j = NBUF * i + b
                gather_wait(b)

                @pl.when(i > 0)
                def _():
                    scatter_wait(b)
                scale(b, j)
                nxt = j + NBUF

                @pl.when(nxt < NCH)
                def _():
                    gather_start(b, nxt)
                scatter_start(b, j)
            return carry
        lax.fori_loop(0, NCH // NBUF, round_body, 0)

        jt = NCH - 1
        gather_wait(0)
        scatter_wait(0)
        scale(0, jt)
        scatter_start(0, jt)
        for b in range(NBUF):
            scatter_wait(b)

        plsc.subcore_barrier()
        out_base = cid * N + s0
        for q in range(RPT // WCH):
            pltpu.sync_copy(acc_sh.at[pl.ds(s0 + q * WCH, WCH)], bounce)
            pltpu.sync_copy(bounce, out_hbm.at[pl.ds(out_base + q * WCH, WCH)])

    return spmm


_spmm64 = _make_spmm(64)
_spmm32 = _make_spmm(32)

_BLK = 1000


def _mm_body(x_ref, w_ref, o_ref):
    o_ref[...] = jnp.dot(x_ref[...], w_ref[...],
                         preferred_element_type=jnp.float32)


def _tc_mm(x, w):
    m, k = x.shape
    n = w.shape[1]
    return pl.pallas_call(
        _mm_body,
        grid=(m // _BLK,),
        in_specs=[pl.BlockSpec((_BLK, k), lambda i: (i, 0)),
                  pl.BlockSpec((k, n), lambda i: (0, 0))],
        out_specs=pl.BlockSpec((_BLK, n), lambda i: (i, 0)),
        out_shape=jax.ShapeDtypeStruct((m, n), jnp.float32),
    )(x, w)


def _leaky(h):
    return jnp.where(h > 0, h, SLOPE * h)


def _combine_mm_body(p_ref, b_ref, w_ref, o_ref):
    h = p_ref[0] + p_ref[1] + b_ref[...]
    h = _leaky(h)
    o_ref[...] = jnp.dot(h, w_ref[...], preferred_element_type=jnp.float32)


def _tc_combine_mm(parts, b, w):
    _, m, k = parts.shape
    n = w.shape[1]
    return pl.pallas_call(
        _combine_mm_body,
        grid=(m // _BLK,),
        in_specs=[pl.BlockSpec((2, _BLK, k), lambda i: (0, i, 0)),
                  pl.BlockSpec((1, k), lambda i: (0, 0)),
                  pl.BlockSpec((k, n), lambda i: (0, 0))],
        out_specs=pl.BlockSpec((_BLK, n), lambda i: (i, 0)),
        out_shape=jax.ShapeDtypeStruct((m, n), jnp.float32),
    )(parts, b, w)


def _head_body(p_ref, b2_ref, wc1_ref, bc1_ref, wc2_ref, bc2_ref, o_ref):
    h = p_ref[0] + p_ref[1] + b2_ref[...]
    h = _leaky(h)
    h = _leaky(jnp.dot(h, wc1_ref[...], preferred_element_type=jnp.float32)
               + bc1_ref[...])
    o_ref[...] = jnp.dot(h, wc2_ref[...],
                         preferred_element_type=jnp.float32) + bc2_ref[...]


def _tc_head(parts, b2, wc1t, bc1, wc2t, bc2):
    _, m, k = parts.shape
    h3 = wc1t.shape[1]
    out = wc2t.shape[1]
    return pl.pallas_call(
        _head_body,
        grid=(m // _BLK,),
        in_specs=[pl.BlockSpec((2, _BLK, k), lambda i: (0, i, 0)),
                  pl.BlockSpec((1, k), lambda i: (0, 0)),
                  pl.BlockSpec((k, h3), lambda i: (0, 0)),
                  pl.BlockSpec((1, h3), lambda i: (0, 0)),
                  pl.BlockSpec((h3, out), lambda i: (0, 0)),
                  pl.BlockSpec((1, out), lambda i: (0, 0))],
        out_specs=pl.BlockSpec((_BLK, out), lambda i: (i, 0)),
        out_shape=jax.ShapeDtypeStruct((m, out), jnp.float32),
    )(parts, b2, wc1t, bc1, wc2t, bc2)


def kernel(x, edge_index, edge_weight, W1, b1, W2, b2, Wc1, bc1, Wc2, bc2):
    packed = jnp.bitwise_or(jnp.left_shift(edge_index[0], 16), edge_index[1])
    packed = packed.reshape(NW * CHUNKS_PER_W, CHUNK)
    w2d = edge_weight.reshape(NW * CHUNKS_PER_W, CHUNK)

    support1 = _tc_mm(x, W1)
    part1 = _spmm64(support1, packed, w2d).reshape(2, N, 64)
    support2 = _tc_combine_mm(part1, b1.reshape(1, -1), W2)
    part2 = _spmm32(support2, packed, w2d).reshape(2, N, 32)
    out = _tc_head(part2, b2.reshape(1, -1), Wc1.T, bc1.reshape(1, -1),
                   Wc2.T, bc2.reshape(1, -1))
    return out


# speedup vs baseline: 15.0311x; 1.0142x over previous
import functools

import jax
import jax.numpy as jnp
from jax import lax
from jax.experimental import pallas as pl
from jax.experimental.pallas import tpu as pltpu
from jax.experimental.pallas import tpu_sc as plsc

N = 10000
E = 320000
SLOPE = 0.2

NC = 2
NS = 16
NW = NC * NS
CHUNK = 80
EDGES_PER_W = E // NW
CHUNKS_PER_W = EDGES_PER_W // CHUNK
RPT = N // NS
WCH = RPT // 5
NBUF = 4
assert CHUNKS_PER_W % NBUF == 1


def _make_spmm(H):
    mesh = plsc.VectorSubcoreMesh(core_axis_name="c", subcore_axis_name="s")
    HB = H // 16
    NCH = CHUNKS_PER_W

    @functools.partial(
        pl.kernel,
        mesh=mesh,
        compiler_params=pltpu.CompilerParams(use_tc_tiling_on_sc=False,
                                             needs_layout_passes=False),
        out_type=jax.ShapeDtypeStruct((2 * N, H), jnp.float32),
        scratch_types=(
            [pltpu.VMEM((NCH, CHUNK), jnp.int32),
             pltpu.VMEM((NCH, CHUNK), jnp.int32),
             pltpu.VMEM((NCH, CHUNK), jnp.float32)]
            + [pltpu.VMEM((CHUNK, H), jnp.bfloat16)] * NBUF
            + [pltpu.VMEM((CHUNK, H), jnp.float32)] * NBUF
            + [pltpu.VMEM((WCH, H), jnp.float32),
               pltpu.VMEM_SHARED((N, H), jnp.float32)]
            + [pltpu.SemaphoreType.DMA] * (2 * NBUF)
        ),
    )
    def spmm(sup_hbm, cr_hbm, w_hbm, out_hbm, col_l, row_l, w_l, *rest):
        gbufs = rest[:NBUF]
        sbufs = rest[NBUF:2 * NBUF]
        bounce = rest[2 * NBUF]
        acc_sh = rest[2 * NBUF + 1]
        gsems = rest[2 * NBUF + 2:3 * NBUF + 2]
        ssems = rest[3 * NBUF + 2:4 * NBUF + 2]
        cid = lax.axis_index("c")
        sid = lax.axis_index("s")
        wid = sid * NC + cid
        s0 = sid * RPT

        wbase = wid * NCH
        pltpu.sync_copy(cr_hbm.at[pl.ds(wbase, NCH)], col_l)
        pltpu.sync_copy(w_hbm.at[pl.ds(wbase, NCH)], w_l)

        def unpack_row(j, carry):
            for g in range(CHUNK // 16):
                sl = pl.ds(g * 16, 16)
                pk = col_l[j, sl]
                row_l[j, sl] = lax.shift_right_logical(pk, 16)
                col_l[j, sl] = lax.bitwise_and(pk, 0xFFFF)
            return carry
        lax.fori_loop(0, NCH, unpack_row, 0)

        def zrow(r, carry):
            for hb in range(HB):
                bounce[r, pl.ds(hb * 16, 16)] = jnp.zeros((16,), jnp.float32)
            return carry
        lax.fori_loop(0, WCH, zrow, 0)
        for q in range(RPT // WCH):
            pltpu.sync_copy(bounce, acc_sh.at[pl.ds(s0 + q * WCH, WCH)])
        plsc.subcore_barrier()

        def gather_start(b, j):
            pltpu.async_copy(sup_hbm.at[col_l.at[j]], gbufs[b], gsems[b])

        def gather_wait(b):
            pltpu.make_async_copy(sup_hbm.at[col_l.at[0]], gbufs[b],
                                  gsems[b]).wait()

        def scatter_start(b, j):
            pltpu.async_copy(sbufs[b], acc_sh.at[row_l.at[j]], ssems[b],
                             add=True)

        def scatter_wait(b):
            pltpu.make_async_copy(sbufs[b], acc_sh.at[row_l.at[0]],
                                  ssems[b]).wait()

        def scale(b, j):
            gb, sb = gbufs[b], sbufs[b]
            for g in range(CHUNK // 16):
                w16 = w_l[j, pl.ds(g * 16, 16)]
                for l in range(16):
                    e = g * 16 + l
                    wb = w16[l]
                    for hb in range(HB // 2):
                        lo, hi = plsc.unpack(
                            gb[e, pl.ds(hb * 32, 32)],
                            format=plsc.PackFormat.INTERLEAVED)
                        sb[e, pl.ds(hb * 32, 16)] = lo * wb
                        sb[e, pl.ds(hb * 32 + 16, 16)] = hi * wb

        for b in range(NBUF):
            gather_start(b, b)

        def round_body(i, carry):
            for b in range(NBUF):
                j = NBUF * i + b
                gather_wait(b)

                @pl.when(i > 0)
                def _():
                    scatter_wait(b)
                scale(b, j)
                nxt = j + NBUF

                @pl.when(nxt < NCH)
                def _():
                    gather_start(b, nxt)
                scatter_start(b, j)
            return carry
        lax.fori_loop(0, NCH // NBUF, round_body, 0)

        jt = NCH - 1
        gather_wait(0)
        scatter_wait(0)
        scale(0, jt)
        scatter_start(0, jt)
        for b in range(NBUF):
            scatter_wait(b)

        plsc.subcore_barrier()
        out_base = cid * N + s0
        for q in range(RPT // WCH):
            pltpu.sync_copy(acc_sh.at[pl.ds(s0 + q * WCH, WCH)], bounce)
            pltpu.sync_copy(bounce, out_hbm.at[pl.ds(out_base + q * WCH, WCH)])

    return spmm


_spmm64 = _make_spmm(64)
_spmm32 = _make_spmm(32)

_BLK = 1000


def _mm_body(x_ref, w_ref, o_ref):
    o_ref[...] = jnp.dot(x_ref[...], w_ref[...],
                         preferred_element_type=jnp.float32
                         ).astype(jnp.bfloat16)


def _tc_mm(x, w):
    m, k = x.shape
    n = w.shape[1]
    return pl.pallas_call(
        _mm_body,
        grid=(m // _BLK,),
        in_specs=[pl.BlockSpec((_BLK, k), lambda i: (i, 0)),
                  pl.BlockSpec((k, n), lambda i: (0, 0))],
        out_specs=pl.BlockSpec((_BLK, n), lambda i: (i, 0)),
        out_shape=jax.ShapeDtypeStruct((m, n), jnp.bfloat16),
    )(x, w)


def _leaky(h):
    return jnp.where(h > 0, h, SLOPE * h)


def _combine_mm_body(p_ref, b_ref, w_ref, o_ref):
    h = p_ref[0] + p_ref[1] + b_ref[...]
    h = _leaky(h)
    o_ref[...] = jnp.dot(h, w_ref[...], preferred_element_type=jnp.float32
                         ).astype(jnp.bfloat16)


def _tc_combine_mm(parts, b, w):
    _, m, k = parts.shape
    n = w.shape[1]
    return pl.pallas_call(
        _combine_mm_body,
        grid=(m // _BLK,),
        in_specs=[pl.BlockSpec((2, _BLK, k), lambda i: (0, i, 0)),
                  pl.BlockSpec((1, k), lambda i: (0, 0)),
                  pl.BlockSpec((k, n), lambda i: (0, 0))],
        out_specs=pl.BlockSpec((_BLK, n), lambda i: (i, 0)),
        out_shape=jax.ShapeDtypeStruct((m, n), jnp.bfloat16),
    )(parts, b, w)


def _head_body(p_ref, b2_ref, wc1_ref, bc1_ref, wc2_ref, bc2_ref, o_ref):
    h = p_ref[0] + p_ref[1] + b2_ref[...]
    h = _leaky(h)
    h = _leaky(jnp.dot(h, wc1_ref[...], preferred_element_type=jnp.float32)
               + bc1_ref[...])
    o_ref[...] = jnp.dot(h, wc2_ref[...],
                         preferred_element_type=jnp.float32) + bc2_ref[...]


def _tc_head(parts, b2, wc1t, bc1, wc2t, bc2):
    _, m, k = parts.shape
    h3 = wc1t.shape[1]
    out = wc2t.shape[1]
    return pl.pallas_call(
        _head_body,
        grid=(m // _BLK,),
        in_specs=[pl.BlockSpec((2, _BLK, k), lambda i: (0, i, 0)),
                  pl.BlockSpec((1, k), lambda i: (0, 0)),
                  pl.BlockSpec((k, h3), lambda i: (0, 0)),
                  pl.BlockSpec((1, h3), lambda i: (0, 0)),
                  pl.BlockSpec((h3, out), lambda i: (0, 0)),
                  pl.BlockSpec((1, out), lambda i: (0, 0))],
        out_specs=pl.BlockSpec((_BLK, out), lambda i: (i, 0)),
        out_shape=jax.ShapeDtypeStruct((m, out), jnp.float32),
    )(parts, b2, wc1t, bc1, wc2t, bc2)


def _interleave_perm(h):
    idx = []
    for g in range(h // 32):
        for k in range(16):
            idx.extend([32 * g + k, 32 * g + 16 + k])
    return jnp.array(idx, dtype=jnp.int32)


def kernel(x, edge_index, edge_weight, W1, b1, W2, b2, Wc1, bc1, Wc2, bc2):
    packed = jnp.bitwise_or(jnp.left_shift(edge_index[0], 16), edge_index[1])
    packed = packed.reshape(NW * CHUNKS_PER_W, CHUNK)
    w2d = edge_weight.reshape(NW * CHUNKS_PER_W, CHUNK)

    support1 = _tc_mm(x, W1[:, _interleave_perm(64)])
    part1 = _spmm64(support1, packed, w2d).reshape(2, N, 64)
    support2 = _tc_combine_mm(part1, b1.reshape(1, -1),
                              W2[:, _interleave_perm(32)])
    part2 = _spmm32(support2, packed, w2d).reshape(2, N, 32)
    out = _tc_head(part2, b2.reshape(1, -1), Wc1.T, bc1.reshape(1, -1),
                   Wc2.T, bc2.reshape(1, -1))
    return out


# speedup vs baseline: 15.6639x; 1.0421x over previous
import functools

import jax
import jax.numpy as jnp
from jax import lax
from jax.experimental import pallas as pl
from jax.experimental.pallas import tpu as pltpu
from jax.experimental.pallas import tpu_sc as plsc

N = 10000
E = 320000
SLOPE = 0.2

NC = 2
NS = 16
NW = NC * NS
CHUNK = 80
EDGES_PER_W = E // NW
CHUNKS_PER_W = EDGES_PER_W // CHUNK
RPT = N // NS
WCH = RPT // 5
NBUF = 4
assert CHUNKS_PER_W % NBUF == 1


def _make_spmm(H):
    mesh = plsc.VectorSubcoreMesh(core_axis_name="c", subcore_axis_name="s")
    HB = H // 16
    NCH = CHUNKS_PER_W

    @functools.partial(
        pl.kernel,
        mesh=mesh,
        compiler_params=pltpu.CompilerParams(use_tc_tiling_on_sc=False,
                                             needs_layout_passes=False),
        out_type=jax.ShapeDtypeStruct((2 * N, H), jnp.float32),
        scratch_types=(
            [pltpu.VMEM((NCH, CHUNK), jnp.int32),
             pltpu.VMEM((NCH, CHUNK), jnp.int32),
             pltpu.VMEM((NCH, CHUNK), jnp.float32)]
            + [pltpu.VMEM((CHUNK, H), jnp.bfloat16)] * NBUF
            + [pltpu.VMEM((CHUNK, H), jnp.float32)] * NBUF
            + [pltpu.VMEM((WCH, H), jnp.float32),
               pltpu.VMEM_SHARED((N, H), jnp.float32)]
            + [pltpu.SemaphoreType.DMA] * (2 * NBUF)
        ),
    )
    def spmm(sup_hbm, cr_hbm, w_hbm, out_hbm, col_l, row_l, w_l, *rest):
        gbufs = rest[:NBUF]
        sbufs = rest[NBUF:2 * NBUF]
        bounce = rest[2 * NBUF]
        acc_sh = rest[2 * NBUF + 1]
        gsems = rest[2 * NBUF + 2:3 * NBUF + 2]
        ssems = rest[3 * NBUF + 2:4 * NBUF + 2]
        cid = lax.axis_index("c")
        sid = lax.axis_index("s")
        wid = sid * NC + cid
        s0 = sid * RPT

        wbase = wid * NCH
        pltpu.sync_copy(cr_hbm.at[pl.ds(wbase, NCH)], col_l)
        pltpu.sync_copy(w_hbm.at[pl.ds(wbase, NCH)], w_l)

        def unpack_row(j, carry):
            for g in range(CHUNK // 16):
                sl = pl.ds(g * 16, 16)
                pk = col_l[j, sl]
                row_l[j, sl] = lax.shift_right_logical(pk, 16)
                col_l[j, sl] = lax.bitwise_and(pk, 0xFFFF)
            return carry
        lax.fori_loop(0, NCH, unpack_row, 0)

        def zrow(r, carry):
            for hb in range(HB):
                bounce[r, pl.ds(hb * 16, 16)] = jnp.zeros((16,), jnp.float32)
            return carry
        lax.fori_loop(0, WCH, zrow, 0)
        for q in range(RPT // WCH):
            pltpu.sync_copy(bounce, acc_sh.at[pl.ds(s0 + q * WCH, WCH)])
        plsc.subcore_barrier()

        def gather_start(b, j):
            pltpu.async_copy(sup_hbm.at[col_l.at[j]], gbufs[b], gsems[b])

        def gather_wait(b):
            pltpu.make_async_copy(sup_hbm.at[col_l.at[0]], gbufs[b],
                                  gsems[b]).wait()

        def scatter_start(b, j):
            pltpu.async_copy(sbufs[b], acc_sh.at[row_l.at[j]], ssems[b],
                             add=True)

        def scatter_wait(b):
            pltpu.make_async_copy(sbufs[b], acc_sh.at[row_l.at[0]],
                                  ssems[b]).wait()

        def scale(b, j):
            gb, sb = gbufs[b], sbufs[b]
            for g in range(CHUNK // 16):
                w16 = w_l[j, pl.ds(g * 16, 16)]
                for l in range(16):
                    e = g * 16 + l
                    wb = w16[l]
                    for hb in range(HB // 2):
                        lo, hi = plsc.unpack(
                            gb[e, pl.ds(hb * 32, 32)],
                            format=plsc.PackFormat.INTERLEAVED)
                        sb[e, pl.ds(hb * 32, 16)] = lo * wb
                        sb[e, pl.ds(hb * 32 + 16, 16)] = hi * wb

        for b in range(NBUF):
            gather_start(b, b)

        def round_body(i, carry):
            for b in range(NBUF):
                j = NBUF * i + b
                gather_wait(b)

                @pl.when(i > 0)
                def _():
                    scatter_wait(b)
                scale(b, j)
                nxt = j + NBUF

                @pl.when(nxt < NCH)
                def _():
                    gather_start(b, nxt)
                scatter_start(b, j)
            return carry
        lax.fori_loop(0, NCH // NBUF, round_body, 0)

        jt = NCH - 1
        gather_wait(0)
        scatter_wait(0)
        scale(0, jt)
        scatter_start(0, jt)
        for b in range(NBUF):
            scatter_wait(b)

        plsc.subcore_barrier()
        out_base = cid * N + s0
        for q in range(RPT // WCH):
            pltpu.sync_copy(acc_sh.at[pl.ds(s0 + q * WCH, WCH)], bounce)
            pltpu.sync_copy(bounce, out_hbm.at[pl.ds(out_base + q * WCH, WCH)])

    return spmm


_spmm64 = _make_spmm(64)
_spmm32 = _make_spmm(32)

_BLK = 2000


def _mm_body(x_ref, w_ref, o_ref):
    o_ref[...] = jnp.dot(x_ref[...], w_ref[...],
                         preferred_element_type=jnp.float32
                         ).astype(jnp.bfloat16)


def _tc_mm(x, w):
    m, k = x.shape
    n = w.shape[1]
    return pl.pallas_call(
        _mm_body,
        grid=(m // _BLK,),
        in_specs=[pl.BlockSpec((_BLK, k), lambda i: (i, 0)),
                  pl.BlockSpec((k, n), lambda i: (0, 0))],
        out_specs=pl.BlockSpec((_BLK, n), lambda i: (i, 0)),
        out_shape=jax.ShapeDtypeStruct((m, n), jnp.bfloat16),
    )(x, w)


def _leaky(h):
    return jnp.where(h > 0, h, SLOPE * h)


def _combine_mm_body(p_ref, b_ref, w_ref, o_ref):
    h = p_ref[0] + p_ref[1] + b_ref[...]
    h = _leaky(h)
    o_ref[...] = jnp.dot(h, w_ref[...], preferred_element_type=jnp.float32
                         ).astype(jnp.bfloat16)


def _tc_combine_mm(parts, b, w):
    _, m, k = parts.shape
    n = w.shape[1]
    return pl.pallas_call(
        _combine_mm_body,
        grid=(m // _BLK,),
        in_specs=[pl.BlockSpec((2, _BLK, k), lambda i: (0, i, 0)),
                  pl.BlockSpec((1, k), lambda i: (0, 0)),
                  pl.BlockSpec((k, n), lambda i: (0, 0))],
        out_specs=pl.BlockSpec((_BLK, n), lambda i: (i, 0)),
        out_shape=jax.ShapeDtypeStruct((m, n), jnp.bfloat16),
    )(parts, b, w)


def _head_body(p_ref, b2_ref, wc1_ref, bc1_ref, wc2_ref, bc2_ref, o_ref):
    h = p_ref[0] + p_ref[1] + b2_ref[...]
    h = _leaky(h)
    h = _leaky(jnp.dot(h, wc1_ref[...], preferred_element_type=jnp.float32)
               + bc1_ref[...])
    o_ref[...] = jnp.dot(h, wc2_ref[...],
                         preferred_element_type=jnp.float32) + bc2_ref[...]


def _tc_head(parts, b2, wc1t, bc1, wc2t, bc2):
    _, m, k = parts.shape
    h3 = wc1t.shape[1]
    out = wc2t.shape[1]
    return pl.pallas_call(
        _head_body,
        grid=(m // _BLK,),
        in_specs=[pl.BlockSpec((2, _BLK, k), lambda i: (0, i, 0)),
                  pl.BlockSpec((1, k), lambda i: (0, 0)),
                  pl.BlockSpec((k, h3), lambda i: (0, 0)),
                  pl.BlockSpec((1, h3), lambda i: (0, 0)),
                  pl.BlockSpec((h3, out), lambda i: (0, 0)),
                  pl.BlockSpec((1, out), lambda i: (0, 0))],
        out_specs=pl.BlockSpec((_BLK, out), lambda i: (i, 0)),
        out_shape=jax.ShapeDtypeStruct((m, out), jnp.float32),
    )(parts, b2, wc1t, bc1, wc2t, bc2)


def _interleave_perm(h):
    idx = []
    for g in range(h // 32):
        for k in range(16):
            idx.extend([32 * g + k, 32 * g + 16 + k])
    return jnp.array(idx, dtype=jnp.int32)


def kernel(x, edge_index, edge_weight, W1, b1, W2, b2, Wc1, bc1, Wc2, bc2):
    packed = jnp.bitwise_or(jnp.left_shift(edge_index[0], 16), edge_index[1])
    packed = packed.reshape(NW * CHUNKS_PER_W, CHUNK)
    w2d = edge_weight.reshape(NW * CHUNKS_PER_W, CHUNK)

    support1 = _tc_mm(x, W1[:, _interleave_perm(64)])
    part1 = _spmm64(support1, packed, w2d).reshape(2, N, 64)
    support2 = _tc_combine_mm(part1, b1.reshape(1, -1),
                              W2[:, _interleave_perm(32)])
    part2 = _spmm32(support2, packed, w2d).reshape(2, N, 32)
    out = _tc_head(part2, b2.reshape(1, -1), Wc1.T, bc1.reshape(1, -1),
                   Wc2.T, bc2.reshape(1, -1))
    return out
